# baseline (speedup 1.0000x reference)
import functools

import jax
import jax.numpy as jnp
from jax import lax
from jax.experimental import pallas as pl
from jax.experimental.pallas import tpu as pltpu
from jax.experimental.pallas import tpu_sc as plsc

VOCAB = 1000000
EMBED = 64
CTX = 200
BATCH = 4096
SEQ = 200

G = 100
N_ROWS = BATCH * SEQ
N_GROUPS = N_ROWS // G
N_WORKERS = 32
GROUPS_PER_W = N_GROUPS // N_WORKERS


def _worker_id():
    return lax.axis_index("s") * 2 + lax.axis_index("c")


def _body(idx_hbm, table_hbm, pos_hbm, out_hbm, pos_v, idx_a, idx_b, buf, sem):
    wid = _worker_id()
    g0 = wid * GROUPS_PER_W

    pltpu.sync_copy(pos_hbm, pos_v)

    def group(g, phase, idx_v):
        pltpu.sync_copy(idx_hbm.at[g], idx_v)
        pltpu.async_copy(table_hbm.at[idx_v], buf, sem).wait()

        def add_row(r, _):
            for k in range(EMBED // 16):
                sl = pl.ds(k * 16, 16)
                buf[r, sl] = buf[r, sl] + pos_v[phase + r, sl]
            return 0

        lax.fori_loop(0, G, add_row, 0, unroll=2)
        pltpu.sync_copy(buf, out_hbm.at[pl.ds(g * G, G)])

    def pair(j, _):
        g = g0 + j * 2
        group(g, 0, idx_a)
        group(g + 1, G, idx_b)
        return 0

    lax.fori_loop(0, GROUPS_PER_W // 2, pair, 0)


@jax.jit
def kernel(token_ids, token_emb, pos_emb):
    idx = token_ids.reshape(N_GROUPS, G).astype(jnp.int32)
    mesh = plsc.VectorSubcoreMesh(core_axis_name="c", subcore_axis_name="s")
    out = pl.kernel(
        _body,
        out_type=jax.ShapeDtypeStruct((N_ROWS, EMBED), jnp.float32),
        mesh=mesh,
        compiler_params=pltpu.CompilerParams(use_tc_tiling_on_sc=False),
        scratch_types=[
            pltpu.VMEM((CTX, EMBED), jnp.float32),
            pltpu.VMEM((G,), jnp.int32),
            pltpu.VMEM((G,), jnp.int32),
            pltpu.VMEM((G, EMBED), jnp.float32),
            pltpu.SemaphoreType.DMA,
        ],
    )(idx, token_emb, pos_emb)
    return out.reshape(BATCH, SEQ, EMBED)


# speedup vs baseline: 1.5947x; 1.5947x over previous
import functools

import jax
import jax.numpy as jnp
from jax import lax
from jax.experimental import pallas as pl
from jax.experimental.pallas import tpu as pltpu
from jax.experimental.pallas import tpu_sc as plsc

VOCAB = 1000000
EMBED = 64
CTX = 200
BATCH = 4096
SEQ = 200

G = 100
K = 8
N_ROWS = BATCH * SEQ
N_GROUPS = N_ROWS // G
N_WORKERS = 32
GROUPS_PER_W = N_GROUPS // N_WORKERS
N_IT = GROUPS_PER_W // K
NK = EMBED // 16


def _worker_id():
    return lax.axis_index("s") * 2 + lax.axis_index("c")


def _body(idx_hbm, table_hbm, pos_hbm, out_hbm, pos_v, idx0, idx1, buf0, buf1,
          gsem0, gsem1, ssem0, ssem1, isem0, isem1):
    wid = _worker_id()
    g0 = wid * GROUPS_PER_W

    pltpu.sync_copy(pos_hbm, pos_v)

    def fire_idx(i, idx, isem):
        off = g0 + jnp.minimum(i, N_IT - 1) * K
        pltpu.async_copy(idx_hbm.at[pl.ds(off, K)], idx, isem)

    def wait_idx(idx, isem):
        pltpu.make_async_copy(idx_hbm.at[pl.ds(0, K)], idx, isem).wait()

    def fire_gathers(idx, buf, gsem):
        for j in range(K):
            pltpu.async_copy(table_hbm.at[idx.at[j]], buf.at[j], gsem)

    def drain_gathers(idx, buf, gsem):
        for j in range(K):
            pltpu.make_async_copy(table_hbm.at[idx.at[j]], buf.at[j], gsem).wait()

    def fire_store(i, buf, ssem):
        pltpu.async_copy(buf, out_hbm.at[pl.ds(g0 + i * K, K)], ssem)

    def wait_store(buf, ssem):
        pltpu.make_async_copy(buf, out_hbm.at[pl.ds(0, K)], ssem).wait()

    def add(buf):
        for phase, jpar in ((0, 0), (G, 1)):
            @plsc.parallel_loop(0, G, 1, unroll=2)
            def _row(r):
                pv = [pos_v[phase + r, pl.ds(k * 16, 16)] for k in range(NK)]
                for p in range(K // 2):
                    j = 2 * p + jpar
                    for k in range(NK):
                        sl = pl.ds(k * 16, 16)
                        buf[j, r, sl] = buf[j, r, sl] + pv[k]

    s0 = (idx0, buf0, gsem0, ssem0, isem0)
    s1 = (idx1, buf1, gsem1, ssem1, isem1)

    def steady(i, X, Y):
        (idxX, bufX, gsemX, ssemX, isemX) = X
        (idxY, bufY, gsemY, ssemY, isemY) = Y
        wait_store(bufY, ssemY)
        wait_idx(idxY, isemY)
        fire_gathers(idxY, bufY, gsemY)
        drain_gathers(idxX, bufX, gsemX)
        fire_idx(i + 2, idxX, isemX)
        add(bufX)
        fire_store(i, bufX, ssemX)

    fire_idx(0, idx0, isem0)
    wait_idx(idx0, isem0)
    fire_gathers(idx0, buf0, gsem0)
    fire_idx(1, idx1, isem1)

    wait_idx(idx1, isem1)
    fire_gathers(idx1, buf1, gsem1)
    drain_gathers(idx0, buf0, gsem0)
    fire_idx(2, idx0, isem0)
    add(buf0)
    fire_store(0, buf0, ssem0)

    def pair(t, _):
        i = 2 * t + 1
        steady(i, s1, s0)
        steady(i + 1, s0, s1)
        return 0

    lax.fori_loop(0, (N_IT - 2) // 2, pair, 0)

    wait_store(buf0, ssem0)
    drain_gathers(idx1, buf1, gsem1)
    add(buf1)
    fire_store(N_IT - 1, buf1, ssem1)
    wait_idx(idx0, isem0)
    wait_store(buf1, ssem1)


@jax.jit
def kernel(token_ids, token_emb, pos_emb):
    idx = token_ids.reshape(N_GROUPS, G).astype(jnp.int32)
    mesh = plsc.VectorSubcoreMesh(core_axis_name="c", subcore_axis_name="s")
    out = pl.kernel(
        _body,
        out_type=jax.ShapeDtypeStruct((N_GROUPS, G, EMBED), jnp.float32),
        mesh=mesh,
        compiler_params=pltpu.CompilerParams(use_tc_tiling_on_sc=False),
        scratch_types=[
            pltpu.VMEM((CTX, EMBED), jnp.float32),
            pltpu.VMEM((K, G), jnp.int32),
            pltpu.VMEM((K, G), jnp.int32),
            pltpu.VMEM((K, G, EMBED), jnp.float32),
            pltpu.VMEM((K, G, EMBED), jnp.float32),
            pltpu.SemaphoreType.DMA,
            pltpu.SemaphoreType.DMA,
            pltpu.SemaphoreType.DMA,
            pltpu.SemaphoreType.DMA,
            pltpu.SemaphoreType.DMA,
            pltpu.SemaphoreType.DMA,
        ],
    )(idx, token_emb, pos_emb)
    return out.reshape(BATCH, SEQ, EMBED)
